# Initial kernel scaffold; baseline (speedup 1.0000x reference)
import jax
import jax.numpy as jnp
from jax.experimental import pallas as pl


def kernel(node_features, edge_indices, edge_features, xbatch, params):
    raise NotImplementedError("write your pallas kernel here")


# speedup vs baseline: 4.3970x; 4.3970x over previous
import functools

import jax
import jax.numpy as jnp
from jax import lax
from jax.experimental import pallas as pl
from jax.experimental.pallas import tpu as pltpu
from jax.experimental.pallas import tpu_sc as plsc

F32 = jnp.float32
I32 = jnp.int32
NUM_MP = 3
LEAK = 0.1
BN_EPS = 1e-5
SC_NC = 2
SC_NS = 16
SC_NW = SC_NC * SC_NS


def _pick(n, cands):
    for c in cands:
        if n % c == 0:
            return c
    return n


def _leaky(x, s):
    return jnp.where(x >= 0, x, s * x)


def _pad_rows(w, m=8):
    r = w.shape[0]
    pr = -r % m
    if pr:
        w = jnp.concatenate([w, jnp.zeros((pr,) + w.shape[1:], w.dtype)], axis=0)
    return w


def _stat_rows(vals, c):
    rid = lax.broadcasted_iota(I32, (8, c), 0)
    u = jnp.zeros((8, c), F32)
    for i, v in enumerate(vals):
        u = jnp.where(rid == i, jnp.broadcast_to(v[None, :], (8, c)), u)
    return u


def _pack_rows(vecs):
    p = jnp.zeros((8, 128), F32)
    for i, v in enumerate(vecs):
        p = p.at[i, : v.shape[0]].set(v.astype(F32))
    return p


def _colstats(x):
    R, C = x.shape
    BR = _pick(R, [6400, 5000, 4000, 2000, 1000, 200, 8])
    grid = R // BR

    def body(x_ref, o_ref):
        @pl.when(pl.program_id(0) == 0)
        def _():
            o_ref[...] = jnp.zeros_like(o_ref)

        xb = x_ref[...]
        o_ref[...] += _stat_rows([jnp.sum(xb, axis=0),
                                  jnp.sum(xb * xb, axis=0)], C)

    return pl.pallas_call(
        body,
        grid=(grid,),
        in_specs=[pl.BlockSpec((BR, C), lambda i: (i, 0))],
        out_specs=pl.BlockSpec((8, C), lambda i: (0, 0)),
        out_shape=jax.ShapeDtypeStruct((8, C), F32),
    )(x)


def _xbn(x, deg8, rgsh):
    R, C = x.shape
    BR = _pick(R, [5000, 4000, 2000, 1000, 200, 8])
    grid = R // BR

    def body(x_ref, d_ref, p_ref, xo_ref, w_ref):
        @pl.when(pl.program_id(0) == 0)
        def _():
            w_ref[...] = jnp.zeros_like(w_ref)

        rg = p_ref[0:1, :C]
        sh = p_ref[1:2, :C]
        xb = x_ref[...] * rg + sh
        xo_ref[...] = xb
        go = d_ref[:, 0:1]
        gi = d_ref[:, 1:2]
        w_ref[...] += _stat_rows([
            jnp.sum(go * xb, axis=0),
            jnp.sum(go * xb * xb, axis=0),
            jnp.sum(gi * xb, axis=0),
            jnp.sum(gi * xb * xb, axis=0),
        ], C)

    return pl.pallas_call(
        body,
        grid=(grid,),
        in_specs=[
            pl.BlockSpec((BR, C), lambda i: (i, 0)),
            pl.BlockSpec((BR, 8), lambda i: (i, 0)),
            pl.BlockSpec((8, 128), lambda i: (0, 0)),
        ],
        out_specs=[
            pl.BlockSpec((BR, C), lambda i: (i, 0)),
            pl.BlockSpec((8, C), lambda i: (0, 0)),
        ],
        out_shape=[
            jax.ShapeDtypeStruct((R, C), F32),
            jax.ShapeDtypeStruct((8, C), F32),
        ],
    )(x, deg8, rgsh)


def _edge_stage1(gsrc, gdst, ev, w1p, gatw, pk):
    E0, nin = gsrc.shape
    ein = ev.shape[1]
    BE = _pick(E0, [6400, 4000, 1600, 800, 400, 80, 8])
    grid = E0 // BE

    def body(gs_ref, gd_ref, e_ref, w1_ref, gw_ref, p_ref, h1_ref, sc_ref, st_ref):
        @pl.when(pl.program_id(0) == 0)
        def _():
            st_ref[...] = jnp.zeros_like(st_ref)

        xs = gs_ref[...]
        xd = gd_ref[...]
        ee = e_ref[...]
        w1 = w1_ref[...]
        z = jnp.dot(xs * p_ref[0:1, :nin] + p_ref[1:2, :nin],
                    w1[0:nin], preferred_element_type=F32)
        z += jnp.dot(xd * p_ref[0:1, nin:2 * nin] + p_ref[1:2, nin:2 * nin],
                     w1[nin:2 * nin], preferred_element_type=F32)
        z += jnp.dot(ee * p_ref[0:1, 2 * nin:2 * nin + ein]
                     + p_ref[1:2, 2 * nin:2 * nin + ein],
                     w1[2 * nin:2 * nin + ein], preferred_element_type=F32)
        z += p_ref[2:3, :32]
        h1 = _leaky(z, LEAK)
        h1_ref[...] = h1
        st_ref[...] += _stat_rows([jnp.sum(h1, axis=0),
                                   jnp.sum(h1 * h1, axis=0)], 32)

        hs = jnp.dot(xs, gw_ref[...], preferred_element_type=F32)
        s = jnp.sum(hs * p_ref[3:4, :32], axis=1, keepdims=True)
        d = jnp.sum(xd * p_ref[4:5, :nin], axis=1, keepdims=True)
        ex = jnp.exp(_leaky(s + d, 0.2))
        sc_ref[...] = jnp.concatenate(
            [hs * ex, ex, jnp.zeros((BE, 7), F32)], axis=1)

    return pl.pallas_call(
        body,
        grid=(grid,),
        in_specs=[
            pl.BlockSpec((BE, nin), lambda i: (i, 0)),
            pl.BlockSpec((BE, nin), lambda i: (i, 0)),
            pl.BlockSpec((BE, ein), lambda i: (i, 0)),
            pl.BlockSpec(w1p.shape, lambda i: (0, 0)),
            pl.BlockSpec((nin, 32), lambda i: (0, 0)),
            pl.BlockSpec((8, 128), lambda i: (0, 0)),
        ],
        out_specs=[
            pl.BlockSpec((BE, 32), lambda i: (i, 0)),
            pl.BlockSpec((BE, 40), lambda i: (i, 0)),
            pl.BlockSpec((8, 32), lambda i: (0, 0)),
        ],
        out_shape=[
            jax.ShapeDtypeStruct((E0, 32), F32),
            jax.ShapeDtypeStruct((E0, 40), F32),
            jax.ShapeDtypeStruct((8, 32), F32),
        ],
    )(gsrc, gdst, ev, w1p, gatw, pk)


def _edge_dense(h, w, pk, slope, with_stats):
    E0, cin = h.shape
    cout = w.shape[1]
    BE = _pick(E0, [6400, 4000, 1600, 800, 400, 80, 8])
    grid = E0 // BE

    def body(h_ref, w_ref, p_ref, o_ref, st_ref=None):
        if st_ref is not None:
            @pl.when(pl.program_id(0) == 0)
            def _():
                st_ref[...] = jnp.zeros_like(st_ref)

        hb = h_ref[...] * p_ref[0:1, :cin] + p_ref[1:2, :cin]
        z = jnp.dot(hb, w_ref[...], preferred_element_type=F32) + p_ref[2:3, :cout]
        if slope is not None:
            z = _leaky(z, slope)
        o_ref[...] = z
        if st_ref is not None:
            st_ref[...] += _stat_rows([jnp.sum(z, axis=0),
                                       jnp.sum(z * z, axis=0)], cout)

    out_specs = [pl.BlockSpec((BE, cout), lambda i: (i, 0))]
    out_shape = [jax.ShapeDtypeStruct((E0, cout), F32)]
    if with_stats:
        out_specs.append(pl.BlockSpec((8, cout), lambda i: (0, 0)))
        out_shape.append(jax.ShapeDtypeStruct((8, cout), F32))
    res = pl.pallas_call(
        body,
        grid=(grid,),
        in_specs=[
            pl.BlockSpec((BE, cin), lambda i: (i, 0)),
            pl.BlockSpec(w.shape, lambda i: (0, 0)),
            pl.BlockSpec((8, 128), lambda i: (0, 0)),
        ],
        out_specs=out_specs,
        out_shape=out_shape,
    )(h, w, pk)
    return res if with_stats else (res[0], None)


def _finalize(acc, xbn, gatw, pk, nodewp):
    HN0, nin = xbn.shape
    BR = _pick(HN0, [5000, 4000, 2000, 1000, 200, 8])
    grid = HN0 // BR

    def body(a_ref, x_ref, gw_ref, p_ref, *rest):
        if nodewp is not None:
            nw_ref, xo_ref, po_ref = rest
        else:
            (xo_ref,) = rest
        xb = x_ref[...]
        h = jnp.dot(xb, gw_ref[...], preferred_element_type=F32)
        s = jnp.sum(h * p_ref[0:1, :32], axis=1, keepdims=True)
        d = jnp.sum(h * p_ref[1:2, :32], axis=1, keepdims=True)
        ex = jnp.exp(_leaky(s + d, 0.2))
        num = a_ref[:, :32] + ex * h
        den = a_ref[:, 32:33] + ex
        xg = num / (den + 1e-16) + p_ref[2:3, :32]
        xn = _leaky(xg, LEAK)
        xo_ref[...] = xn
        if nodewp is not None:
            po_ref[...] = (jnp.dot(xn, nw_ref[...], preferred_element_type=F32)
                           + p_ref[3:4, :8])

    in_specs = [
        pl.BlockSpec((BR, 40), lambda i: (i, 0)),
        pl.BlockSpec((BR, nin), lambda i: (i, 0)),
        pl.BlockSpec((nin, 32), lambda i: (0, 0)),
        pl.BlockSpec((8, 128), lambda i: (0, 0)),
    ]
    args = [acc, xbn, gatw, pk]
    out_specs = [pl.BlockSpec((BR, 32), lambda i: (i, 0))]
    out_shape = [jax.ShapeDtypeStruct((HN0, 32), F32)]
    if nodewp is not None:
        in_specs.append(pl.BlockSpec((32, 8), lambda i: (0, 0)))
        args.append(nodewp)
        out_specs.append(pl.BlockSpec((BR, 8), lambda i: (i, 0)))
        out_shape.append(jax.ShapeDtypeStruct((HN0, 8), F32))
    res = pl.pallas_call(
        body,
        grid=(grid,),
        in_specs=in_specs,
        out_specs=out_specs,
        out_shape=out_shape,
    )(*args)
    return res if nodewp is not None else (res[0], None)


def _remap(src2, dst2, hq, trash):
    R, C = src2.shape
    BR = _pick(R, [8, 4, 2, 1])
    grid = R // BR

    def body(s_ref, d_ref, *outs):
        s = s_ref[...]
        d = d_ref[...]
        for q in range(4):
            lo, hi = q * hq, (q + 1) * hq
            outs[q][...] = jnp.where((s >= lo) & (s < hi), s - lo, trash)
            outs[4 + q][...] = jnp.where((d >= lo) & (d < hi), d - lo, trash)

    spec = pl.BlockSpec((BR, C), lambda i: (i, 0))
    sh = jax.ShapeDtypeStruct((R, C), I32)
    return pl.pallas_call(
        body,
        grid=(grid,),
        in_specs=[spec, spec],
        out_specs=[spec] * 8,
        out_shape=[sh] * 8,
    )(src2, dst2)


@functools.lru_cache(maxsize=None)
def _sc_gather_kernel(E0, C):
    ew = E0 // SC_NW
    gb = _pick(ew, [200, 8])
    iters = ew // gb
    mesh = plsc.VectorSubcoreMesh(core_axis_name="c", subcore_axis_name="s")

    @functools.partial(
        pl.kernel,
        mesh=mesh,
        compiler_params=pltpu.CompilerParams(use_tc_tiling_on_sc=False),
        out_type=[
            jax.ShapeDtypeStruct((E0, C), F32),
            jax.ShapeDtypeStruct((E0, C), F32),
        ],
        scratch_types=[
            pltpu.VMEM((gb,), I32),
            pltpu.VMEM((gb, C), F32),
            pltpu.SemaphoreType.DMA,
        ],
    )
    def k(x_hbm, s_hbm, d_hbm, gs_hbm, gd_hbm, idx_v, rows_v, sem):
        wid = lax.axis_index("s") * SC_NC + lax.axis_index("c")
        base = wid * ew

        @pl.loop(0, iters)
        def _(i):
            off = base + i * gb
            pltpu.sync_copy(s_hbm.at[pl.ds(off, gb)], idx_v)
            pltpu.async_copy(x_hbm.at[idx_v], rows_v, sem).wait()
            pltpu.sync_copy(rows_v, gs_hbm.at[pl.ds(off, gb)])
            pltpu.sync_copy(d_hbm.at[pl.ds(off, gb)], idx_v)
            pltpu.async_copy(x_hbm.at[idx_v], rows_v, sem).wait()
            pltpu.sync_copy(rows_v, gd_hbm.at[pl.ds(off, gb)])

    return k


def _sc_gather(xbn, src, dst):
    return _sc_gather_kernel(src.shape[0], xbn.shape[1])(xbn, src, dst)


@functools.lru_cache(maxsize=None)
def _sc_scatter_kernel(E0, W, hp):
    es = E0 // SC_NS
    sb = _pick(es, [200, 8])
    iters = es // sb
    rps = hp // SC_NS
    mesh = plsc.VectorSubcoreMesh(core_axis_name="c", subcore_axis_name="s")

    @functools.partial(
        pl.kernel,
        mesh=mesh,
        compiler_params=pltpu.CompilerParams(use_tc_tiling_on_sc=False),
        out_type=[
            jax.ShapeDtypeStruct((hp, W), F32),
            jax.ShapeDtypeStruct((hp, W), F32),
        ],
        scratch_types=[
            pltpu.VMEM((sb,), I32),
            pltpu.VMEM((sb, W), F32),
            pltpu.VMEM_SHARED((hp, W), F32),
        ],
    )
    def k(r_hbm, i0_hbm, i1_hbm, z_hbm, a0_hbm, a1_hbm, idx_v, rows_v, shared):
        cid = lax.axis_index("c")
        sid = lax.axis_index("s")

        @pl.when(sid == 0)
        def _():
            pltpu.sync_copy(z_hbm, shared)

        plsc.subcore_barrier()

        def run(ix_hbm):
            @pl.loop(0, iters)
            def _(i):
                off = sid * es + i * sb
                pltpu.sync_copy(ix_hbm.at[pl.ds(off, sb)], idx_v)
                pltpu.sync_copy(r_hbm.at[pl.ds(off, sb)], rows_v)
                pltpu.sync_copy(rows_v, shared.at[idx_v], add=True)

        @pl.when(cid == 0)
        def _():
            run(i0_hbm)

        @pl.when(cid == 1)
        def _():
            run(i1_hbm)

        plsc.subcore_barrier()

        @pl.when(cid == 0)
        def _():
            pltpu.sync_copy(shared.at[pl.ds(sid * rps, rps)],
                            a0_hbm.at[pl.ds(sid * rps, rps)])

        @pl.when(cid == 1)
        def _():
            pltpu.sync_copy(shared.at[pl.ds(sid * rps, rps)],
                            a1_hbm.at[pl.ds(sid * rps, rps)])

    return k


def _sc_scatter(rows, idx0, idx1, zrows, hp):
    return _sc_scatter_kernel(rows.shape[0], rows.shape[1], hp)(
        rows, idx0, idx1, zrows)


def _moments(s1, s2, n):
    mu = s1 / n
    var = s2 / n - mu * mu
    return mu, var


def _affine(mu, var, g, b):
    rg = lax.rsqrt(var + BN_EPS) * g
    return rg, b - mu * rg


def kernel(node_features, edge_indices, edge_features, xbatch, params):
    del xbatch
    n = node_features.shape[0]
    e_cnt = edge_indices.shape[1]
    hq = n // 4
    rps = -(-(hq + 8) // SC_NS)
    rps += -rps % 8
    hp = rps * SC_NS
    src = edge_indices[0]
    dst = edge_indices[1]

    cols = _pick(e_cnt, [8000, 4000, 2000, 1000, 8])
    qs = _remap(src.reshape(-1, cols), dst.reshape(-1, cols), hq, hq)
    sq = [a.reshape(-1) for a in qs[:4]]
    dq = [a.reshape(-1) for a in qs[4:]]

    tok = jnp.zeros((1, 1), F32)

    def scatter4(rows, idx4, z):
        nonlocal tok
        a0, a1 = _sc_scatter(rows, idx4[0], idx4[1], z + tok, hp)
        a2, a3 = _sc_scatter(rows, idx4[2], idx4[3], z + a0[0:1, 0:1] * 0.0, hp)
        tok = a2[0:1, 0:1] * 0.0
        return a0, a1, a2, a3

    ones8 = jnp.concatenate(
        [jnp.ones((e_cnt, 1), F32), jnp.zeros((e_cnt, 7), F32)], axis=1)
    z8 = jnp.zeros((hp, 8), F32)
    do = scatter4(ones8, sq, z8)
    di = scatter4(ones8, dq, z8)
    deg8 = jnp.concatenate([
        jnp.concatenate([a[:hq, 0:1] for a in do], axis=0),
        jnp.concatenate([a[:hq, 0:1] for a in di], axis=0),
        jnp.zeros((n, 6), F32),
    ], axis=1)

    z40 = jnp.zeros((hp, 40), F32)
    est = _colstats(edge_features)

    x = node_features
    ev = edge_features
    x_pred = None
    e_pred = None
    for r in range(NUM_MP):
        p = params['mp%d' % r]
        nin = x.shape[1]
        ein = ev.shape[1]

        nst = _colstats(x)
        mu, var = _moments(nst[0, :nin], nst[1, :nin], n)
        rg, sh = _affine(mu, var, p['bn_node_g'], p['bn_node_b'])
        xbn, wst = _xbn(x, deg8, _pack_rows([rg, sh]))

        mu_cat, var_cat = _moments(
            jnp.concatenate([wst[0, :nin], wst[2, :nin], est[0, :ein]]),
            jnp.concatenate([wst[1, :nin], wst[3, :nin], est[1, :ein]]),
            float(e_cnt))
        rg1, sh1 = _affine(mu_cat, var_cat, p['bn1_g'], p['bn1_b'])

        gsrc, gdst = _sc_gather(xbn, src, dst)

        v_d = p['gat_W'] @ p['gat_adst']
        pk1 = _pack_rows([rg1, sh1, p['b1'], p['gat_asrc'], v_d])
        h1, scat, st1 = _edge_stage1(gsrc, gdst, ev, _pad_rows(p['w1']),
                                     p['gat_W'], pk1)

        mu2, var2 = _moments(st1[0, :32], st1[1, :32], float(e_cnt))
        rg2, sh2 = _affine(mu2, var2, p['bn2_g'], p['bn2_b'])
        h2, st2 = _edge_dense(h1, p['w2'], _pack_rows([rg2, sh2, p['b2']]),
                              LEAK, True)

        mu3, var3 = _moments(st2[0, :32], st2[1, :32], float(e_cnt))
        rg3, sh3 = _affine(mu3, var3, p['bn3_g'], p['bn3_b'])
        if r < NUM_MP - 1:
            ev, est = _edge_dense(h2, p['w3'], _pack_rows([rg3, sh3, p['b3']]),
                                  None, True)
        else:
            w3e = _pad_rows((p['w3'] @ params['edge_W']).T, 8).T
            b3e = p['b3'] @ params['edge_W'] + params['edge_b']
            ep8, _ = _edge_dense(h2, w3e, _pack_rows([rg3, sh3, b3e]),
                                 None, False)
            e_pred = ep8[:, :2]

        accs = scatter4(scat, dq, z40)

        nodewp = None
        pkf = [p['gat_asrc'], p['gat_adst'], p['gat_bias']]
        if r == NUM_MP - 1:
            nodewp = _pad_rows(params['node_W'].T, 8).T
            pkf.append(params['node_b'])
        pkf = _pack_rows(pkf)
        fins = [_finalize(accs[q][:hq], xbn[q * hq:(q + 1) * hq],
                          p['gat_W'], pkf, nodewp) for q in range(4)]
        x = jnp.concatenate([f[0] for f in fins], axis=0)
        if r == NUM_MP - 1:
            x_pred = jnp.concatenate([f[1][:, :2] for f in fins], axis=0)

    return x_pred, e_pred


# speedup vs baseline: 5.2782x; 1.2004x over previous
import functools

import jax
import jax.numpy as jnp
from jax import lax
from jax.experimental import pallas as pl
from jax.experimental.pallas import tpu as pltpu
from jax.experimental.pallas import tpu_sc as plsc

F32 = jnp.float32
I32 = jnp.int32
NUM_MP = 3
LEAK = 0.1
BN_EPS = 1e-5
SC_NC = 2
SC_NS = 16
SC_NW = SC_NC * SC_NS


def _pick(n, cands):
    for c in cands:
        if n % c == 0:
            return c
    return n


def _leaky(x, s):
    return jnp.where(x >= 0, x, s * x)


def _pad_rows(w, m=8):
    r = w.shape[0]
    pr = -r % m
    if pr:
        w = jnp.concatenate([w, jnp.zeros((pr,) + w.shape[1:], w.dtype)], axis=0)
    return w


def _stat_rows(vals, c):
    rid = lax.broadcasted_iota(I32, (8, c), 0)
    u = jnp.zeros((8, c), F32)
    for i, v in enumerate(vals):
        u = jnp.where(rid == i, jnp.broadcast_to(v[None, :], (8, c)), u)
    return u


def _pack_rows(vecs):
    p = jnp.zeros((8, 128), F32)
    for i, v in enumerate(vecs):
        p = p.at[i, : v.shape[0]].set(v.astype(F32))
    return p


def _colstats(x):
    R, C = x.shape
    BR = _pick(R, [6400, 5000, 4000, 2000, 1000, 200, 8])
    grid = R // BR

    def body(x_ref, o_ref):
        @pl.when(pl.program_id(0) == 0)
        def _():
            o_ref[...] = jnp.zeros_like(o_ref)

        xb = x_ref[...]
        o_ref[...] += _stat_rows([jnp.sum(xb, axis=0),
                                  jnp.sum(xb * xb, axis=0)], C)

    return pl.pallas_call(
        body,
        grid=(grid,),
        in_specs=[pl.BlockSpec((BR, C), lambda i: (i, 0))],
        out_specs=pl.BlockSpec((8, C), lambda i: (0, 0)),
        out_shape=jax.ShapeDtypeStruct((8, C), F32),
    )(x)


def _xbn(x, rgsh):
    R, C = x.shape
    BR = _pick(R, [5000, 4000, 2000, 1000, 200, 8])
    grid = R // BR

    def body(x_ref, p_ref, xo_ref):
        xo_ref[...] = x_ref[...] * p_ref[0:1, :C] + p_ref[1:2, :C]

    return pl.pallas_call(
        body,
        grid=(grid,),
        in_specs=[
            pl.BlockSpec((BR, C), lambda i: (i, 0)),
            pl.BlockSpec((8, 128), lambda i: (0, 0)),
        ],
        out_specs=pl.BlockSpec((BR, C), lambda i: (i, 0)),
        out_shape=jax.ShapeDtypeStruct((R, C), F32),
    )(x, rgsh)


def _edge_stage1(gsrc, gdst, ev, w1p, gatw, pk):
    E0, nin = gsrc.shape
    ein = ev.shape[1]
    BE = _pick(E0, [6400, 4000, 1600, 800, 400, 80, 8])
    grid = E0 // BE

    def body(gs_ref, gd_ref, e_ref, w1_ref, gw_ref, p_ref, h1_ref, sc_ref, st_ref):
        @pl.when(pl.program_id(0) == 0)
        def _():
            st_ref[...] = jnp.zeros_like(st_ref)

        xs = gs_ref[...]
        xd = gd_ref[...]
        ee = e_ref[...]
        w1 = w1_ref[...]
        z = jnp.dot(xs * p_ref[0:1, :nin] + p_ref[1:2, :nin],
                    w1[0:nin], preferred_element_type=F32)
        z += jnp.dot(xd * p_ref[0:1, nin:2 * nin] + p_ref[1:2, nin:2 * nin],
                     w1[nin:2 * nin], preferred_element_type=F32)
        z += jnp.dot(ee * p_ref[0:1, 2 * nin:2 * nin + ein]
                     + p_ref[1:2, 2 * nin:2 * nin + ein],
                     w1[2 * nin:2 * nin + ein], preferred_element_type=F32)
        z += p_ref[2:3, :32]
        h1 = _leaky(z, LEAK)
        h1_ref[...] = h1
        st_ref[...] += _stat_rows([jnp.sum(h1, axis=0),
                                   jnp.sum(h1 * h1, axis=0)], 32)

        hs = jnp.dot(xs, gw_ref[...], preferred_element_type=F32)
        s = jnp.sum(hs * p_ref[3:4, :32], axis=1, keepdims=True)
        d = jnp.sum(xd * p_ref[4:5, :nin], axis=1, keepdims=True)
        ex = jnp.exp(_leaky(s + d, 0.2))
        sc_ref[...] = jnp.concatenate(
            [hs * ex, ex, jnp.zeros((BE, 7), F32)], axis=1)

    return pl.pallas_call(
        body,
        grid=(grid,),
        in_specs=[
            pl.BlockSpec((BE, nin), lambda i: (i, 0)),
            pl.BlockSpec((BE, nin), lambda i: (i, 0)),
            pl.BlockSpec((BE, ein), lambda i: (i, 0)),
            pl.BlockSpec(w1p.shape, lambda i: (0, 0)),
            pl.BlockSpec((nin, 32), lambda i: (0, 0)),
            pl.BlockSpec((8, 128), lambda i: (0, 0)),
        ],
        out_specs=[
            pl.BlockSpec((BE, 32), lambda i: (i, 0)),
            pl.BlockSpec((BE, 40), lambda i: (i, 0)),
            pl.BlockSpec((8, 32), lambda i: (0, 0)),
        ],
        out_shape=[
            jax.ShapeDtypeStruct((E0, 32), F32),
            jax.ShapeDtypeStruct((E0, 40), F32),
            jax.ShapeDtypeStruct((8, 32), F32),
        ],
    )(gsrc, gdst, ev, w1p, gatw, pk)


def _edge_dense(h, w, pk, slope, with_stats):
    E0, cin = h.shape
    cout = w.shape[1]
    BE = _pick(E0, [6400, 4000, 1600, 800, 400, 80, 8])
    grid = E0 // BE

    def body(h_ref, w_ref, p_ref, o_ref, st_ref=None):
        if st_ref is not None:
            @pl.when(pl.program_id(0) == 0)
            def _():
                st_ref[...] = jnp.zeros_like(st_ref)

        hb = h_ref[...] * p_ref[0:1, :cin] + p_ref[1:2, :cin]
        z = jnp.dot(hb, w_ref[...], preferred_element_type=F32) + p_ref[2:3, :cout]
        if slope is not None:
            z = _leaky(z, slope)
        o_ref[...] = z
        if st_ref is not None:
            st_ref[...] += _stat_rows([jnp.sum(z, axis=0),
                                       jnp.sum(z * z, axis=0)], cout)

    out_specs = [pl.BlockSpec((BE, cout), lambda i: (i, 0))]
    out_shape = [jax.ShapeDtypeStruct((E0, cout), F32)]
    if with_stats:
        out_specs.append(pl.BlockSpec((8, cout), lambda i: (0, 0)))
        out_shape.append(jax.ShapeDtypeStruct((8, cout), F32))
    res = pl.pallas_call(
        body,
        grid=(grid,),
        in_specs=[
            pl.BlockSpec((BE, cin), lambda i: (i, 0)),
            pl.BlockSpec(w.shape, lambda i: (0, 0)),
            pl.BlockSpec((8, 128), lambda i: (0, 0)),
        ],
        out_specs=out_specs,
        out_shape=out_shape,
    )(h, w, pk)
    return res if with_stats else (res[0], None)


def _finalize(acc, xbn, gatw, pk, nodewp):
    HN0, nin = xbn.shape
    BR = _pick(HN0, [5000, 4000, 2000, 1000, 200, 8])
    grid = HN0 // BR

    def body(a_ref, x_ref, gw_ref, p_ref, *rest):
        if nodewp is not None:
            nw_ref, xo_ref, po_ref = rest
        else:
            (xo_ref,) = rest
        xb = x_ref[...]
        h = jnp.dot(xb, gw_ref[...], preferred_element_type=F32)
        s = jnp.sum(h * p_ref[0:1, :32], axis=1, keepdims=True)
        d = jnp.sum(h * p_ref[1:2, :32], axis=1, keepdims=True)
        ex = jnp.exp(_leaky(s + d, 0.2))
        num = a_ref[:, :32] + ex * h
        den = a_ref[:, 32:33] + ex
        xg = num / (den + 1e-16) + p_ref[2:3, :32]
        xn = _leaky(xg, LEAK)
        xo_ref[...] = xn
        if nodewp is not None:
            po_ref[...] = (jnp.dot(xn, nw_ref[...], preferred_element_type=F32)
                           + p_ref[3:4, :8])

    in_specs = [
        pl.BlockSpec((BR, 40), lambda i: (i, 0)),
        pl.BlockSpec((BR, nin), lambda i: (i, 0)),
        pl.BlockSpec((nin, 32), lambda i: (0, 0)),
        pl.BlockSpec((8, 128), lambda i: (0, 0)),
    ]
    args = [acc, xbn, gatw, pk]
    out_specs = [pl.BlockSpec((BR, 32), lambda i: (i, 0))]
    out_shape = [jax.ShapeDtypeStruct((HN0, 32), F32)]
    if nodewp is not None:
        in_specs.append(pl.BlockSpec((32, 8), lambda i: (0, 0)))
        args.append(nodewp)
        out_specs.append(pl.BlockSpec((BR, 8), lambda i: (i, 0)))
        out_shape.append(jax.ShapeDtypeStruct((HN0, 8), F32))
    res = pl.pallas_call(
        body,
        grid=(grid,),
        in_specs=in_specs,
        out_specs=out_specs,
        out_shape=out_shape,
    )(*args)
    return res if nodewp is not None else (res[0], None)


def _remap(dst2, hq, trash):
    R, C = dst2.shape
    BR = _pick(R, [8, 4, 2, 1])
    grid = R // BR

    def body(d_ref, *outs):
        d = d_ref[...]
        for q in range(4):
            lo, hi = q * hq, (q + 1) * hq
            outs[q][...] = jnp.where((d >= lo) & (d < hi), d - lo, trash)

    spec = pl.BlockSpec((BR, C), lambda i: (i, 0))
    sh = jax.ShapeDtypeStruct((R, C), I32)
    return pl.pallas_call(
        body,
        grid=(grid,),
        in_specs=[spec],
        out_specs=[spec] * 4,
        out_shape=[sh] * 4,
    )(dst2)


@functools.lru_cache(maxsize=None)
def _sc_gather_kernel(E0, C):
    ew = E0 // SC_NW
    gb = _pick(ew, [400, 200, 8])
    iters = ew // gb
    mesh = plsc.VectorSubcoreMesh(core_axis_name="c", subcore_axis_name="s")

    @functools.partial(
        pl.kernel,
        mesh=mesh,
        compiler_params=pltpu.CompilerParams(use_tc_tiling_on_sc=False),
        out_type=[
            jax.ShapeDtypeStruct((E0, C), F32),
            jax.ShapeDtypeStruct((E0, C), F32),
        ],
        scratch_types=[
            pltpu.VMEM((gb,), I32),
            pltpu.VMEM((gb, C), F32),
            pltpu.SemaphoreType.DMA,
        ],
    )
    def k(x_hbm, s_hbm, d_hbm, gs_hbm, gd_hbm, idx_v, rows_v, sem):
        wid = lax.axis_index("s") * SC_NC + lax.axis_index("c")
        base = wid * ew

        @pl.loop(0, iters)
        def _(i):
            off = base + i * gb
            pltpu.sync_copy(s_hbm.at[pl.ds(off, gb)], idx_v)
            pltpu.async_copy(x_hbm.at[idx_v], rows_v, sem).wait()
            pltpu.sync_copy(rows_v, gs_hbm.at[pl.ds(off, gb)])
            pltpu.sync_copy(d_hbm.at[pl.ds(off, gb)], idx_v)
            pltpu.async_copy(x_hbm.at[idx_v], rows_v, sem).wait()
            pltpu.sync_copy(rows_v, gd_hbm.at[pl.ds(off, gb)])

    return k


def _sc_gather(xbn, src, dst):
    return _sc_gather_kernel(src.shape[0], xbn.shape[1])(xbn, src, dst)


@functools.lru_cache(maxsize=None)
def _sc_scatter_kernel(E0, W, hp):
    es = E0 // SC_NS
    sb = _pick(es, [400, 200, 8])
    iters = es // sb
    rps = hp // SC_NS
    mesh = plsc.VectorSubcoreMesh(core_axis_name="c", subcore_axis_name="s")

    @functools.partial(
        pl.kernel,
        mesh=mesh,
        compiler_params=pltpu.CompilerParams(use_tc_tiling_on_sc=False),
        out_type=[
            jax.ShapeDtypeStruct((hp, W), F32),
            jax.ShapeDtypeStruct((hp, W), F32),
        ],
        scratch_types=[
            pltpu.VMEM((sb,), I32),
            pltpu.VMEM((sb, W), F32),
            pltpu.VMEM_SHARED((hp, W), F32),
        ],
    )
    def k(r_hbm, i0_hbm, i1_hbm, z_hbm, a0_hbm, a1_hbm, idx_v, rows_v, shared):
        cid = lax.axis_index("c")
        sid = lax.axis_index("s")

        @pl.when(sid == 0)
        def _():
            pltpu.sync_copy(z_hbm, shared)

        plsc.subcore_barrier()

        def run(ix_hbm):
            @pl.loop(0, iters)
            def _(i):
                off = sid * es + i * sb
                pltpu.sync_copy(ix_hbm.at[pl.ds(off, sb)], idx_v)
                pltpu.sync_copy(r_hbm.at[pl.ds(off, sb)], rows_v)
                pltpu.sync_copy(rows_v, shared.at[idx_v], add=True)

        @pl.when(cid == 0)
        def _():
            run(i0_hbm)

        @pl.when(cid == 1)
        def _():
            run(i1_hbm)

        plsc.subcore_barrier()

        @pl.when(cid == 0)
        def _():
            pltpu.sync_copy(shared.at[pl.ds(sid * rps, rps)],
                            a0_hbm.at[pl.ds(sid * rps, rps)])

        @pl.when(cid == 1)
        def _():
            pltpu.sync_copy(shared.at[pl.ds(sid * rps, rps)],
                            a1_hbm.at[pl.ds(sid * rps, rps)])

    return k


def _sc_scatter(rows, idx0, idx1, zrows, hp):
    return _sc_scatter_kernel(rows.shape[0], rows.shape[1], hp)(
        rows, idx0, idx1, zrows)


def _moments(s1, s2, n):
    mu = s1 / n
    var = s2 / n - mu * mu
    return mu, var


def _affine(mu, var, g, b):
    rg = lax.rsqrt(var + BN_EPS) * g
    return rg, b - mu * rg


def kernel(node_features, edge_indices, edge_features, xbatch, params):
    del xbatch
    n = node_features.shape[0]
    e_cnt = edge_indices.shape[1]
    hq = n // 4
    rps = -(-(hq + 8) // SC_NS)
    rps += -rps % 8
    hp = rps * SC_NS
    src = edge_indices[0]
    dst = edge_indices[1]

    cols = _pick(e_cnt, [8000, 4000, 2000, 1000, 8])
    dq = [a.reshape(-1) for a in _remap(dst.reshape(-1, cols), hq, hq)]

    def scatter4(rows, idx4, z):
        a0, a1 = _sc_scatter(rows, idx4[0], idx4[1], z, hp)
        a2, a3 = _sc_scatter(rows, idx4[2], idx4[3], z + a0[0:1, 0:1] * 0.0, hp)
        return a0, a1, a2, a3

    z40 = jnp.zeros((hp, 40), F32)
    est = _colstats(edge_features)

    x = node_features
    ev = edge_features
    x_pred = None
    e_pred = None
    for r in range(NUM_MP):
        p = params['mp%d' % r]
        nin = x.shape[1]
        ein = ev.shape[1]

        nst = _colstats(x)
        mu, var = _moments(nst[0, :nin], nst[1, :nin], n)
        rg, sh = _affine(mu, var, p['bn_node_g'], p['bn_node_b'])
        xbn = _xbn(x, _pack_rows([rg, sh]))

        gsrc, gdst = _sc_gather(xbn, src, dst)

        sst = _colstats(gsrc)
        dstt = _colstats(gdst)
        mu_cat, var_cat = _moments(
            jnp.concatenate([sst[0, :nin], dstt[0, :nin], est[0, :ein]]),
            jnp.concatenate([sst[1, :nin], dstt[1, :nin], est[1, :ein]]),
            float(e_cnt))
        rg1, sh1 = _affine(mu_cat, var_cat, p['bn1_g'], p['bn1_b'])

        v_d = p['gat_W'] @ p['gat_adst']
        pk1 = _pack_rows([rg1, sh1, p['b1'], p['gat_asrc'], v_d])
        h1, scat, st1 = _edge_stage1(gsrc, gdst, ev, _pad_rows(p['w1']),
                                     p['gat_W'], pk1)

        mu2, var2 = _moments(st1[0, :32], st1[1, :32], float(e_cnt))
        rg2, sh2 = _affine(mu2, var2, p['bn2_g'], p['bn2_b'])
        h2, st2 = _edge_dense(h1, p['w2'], _pack_rows([rg2, sh2, p['b2']]),
                              LEAK, True)

        mu3, var3 = _moments(st2[0, :32], st2[1, :32], float(e_cnt))
        rg3, sh3 = _affine(mu3, var3, p['bn3_g'], p['bn3_b'])
        if r < NUM_MP - 1:
            ev, est = _edge_dense(h2, p['w3'], _pack_rows([rg3, sh3, p['b3']]),
                                  None, True)
        else:
            w3e = _pad_rows((p['w3'] @ params['edge_W']).T, 8).T
            b3e = p['b3'] @ params['edge_W'] + params['edge_b']
            ep8, _ = _edge_dense(h2, w3e, _pack_rows([rg3, sh3, b3e]),
                                 None, False)
            e_pred = ep8[:, :2]

        accs = scatter4(scat, dq, z40)

        nodewp = None
        pkf = [p['gat_asrc'], p['gat_adst'], p['gat_bias']]
        if r == NUM_MP - 1:
            nodewp = _pad_rows(params['node_W'].T, 8).T
            pkf.append(params['node_b'])
        pkf = _pack_rows(pkf)
        fins = [_finalize(accs[q][:hq], xbn[q * hq:(q + 1) * hq],
                          p['gat_W'], pkf, nodewp) for q in range(4)]
        x = jnp.concatenate([f[0] for f in fins], axis=0)
        if r == NUM_MP - 1:
            x_pred = jnp.concatenate([f[1][:, :2] for f in fins], axis=0)

    return x_pred, e_pred


# speedup vs baseline: 5.2921x; 1.0026x over previous
import functools

import jax
import jax.numpy as jnp
from jax import lax
from jax.experimental import pallas as pl
from jax.experimental.pallas import tpu as pltpu
from jax.experimental.pallas import tpu_sc as plsc

F32 = jnp.float32
I32 = jnp.int32
NUM_MP = 3
LEAK = 0.1
BN_EPS = 1e-5
SC_NC = 2
SC_NS = 16
SC_NW = SC_NC * SC_NS


def _pick(n, cands):
    for c in cands:
        if n % c == 0:
            return c
    return n


def _leaky(x, s):
    return jnp.where(x >= 0, x, s * x)


def _pad_rows(w, m=8):
    r = w.shape[0]
    pr = -r % m
    if pr:
        w = jnp.concatenate([w, jnp.zeros((pr,) + w.shape[1:], w.dtype)], axis=0)
    return w


def _stat_rows(vals, c):
    rid = lax.broadcasted_iota(I32, (8, c), 0)
    u = jnp.zeros((8, c), F32)
    for i, v in enumerate(vals):
        u = jnp.where(rid == i, jnp.broadcast_to(v[None, :], (8, c)), u)
    return u


def _pack_rows(vecs):
    p = jnp.zeros((8, 128), F32)
    for i, v in enumerate(vecs):
        p = p.at[i, : v.shape[0]].set(v.astype(F32))
    return p


def _colstats(x):
    R, C = x.shape
    BR = _pick(R, [6400, 5000, 4000, 2000, 1000, 200, 8])
    grid = R // BR

    def body(x_ref, o_ref):
        @pl.when(pl.program_id(0) == 0)
        def _():
            o_ref[...] = jnp.zeros_like(o_ref)

        xb = x_ref[...]
        o_ref[...] += _stat_rows([jnp.sum(xb, axis=0),
                                  jnp.sum(xb * xb, axis=0)], C)

    return pl.pallas_call(
        body,
        grid=(grid,),
        in_specs=[pl.BlockSpec((BR, C), lambda i: (i, 0))],
        out_specs=pl.BlockSpec((8, C), lambda i: (0, 0)),
        out_shape=jax.ShapeDtypeStruct((8, C), F32),
    )(x)


def _xbn(x, rgsh):
    R, C = x.shape
    BR = _pick(R, [5000, 4000, 2000, 1000, 200, 8])
    grid = R // BR

    def body(x_ref, p_ref, xo_ref):
        xo_ref[...] = x_ref[...] * p_ref[0:1, :C] + p_ref[1:2, :C]

    return pl.pallas_call(
        body,
        grid=(grid,),
        in_specs=[
            pl.BlockSpec((BR, C), lambda i: (i, 0)),
            pl.BlockSpec((8, 128), lambda i: (0, 0)),
        ],
        out_specs=pl.BlockSpec((BR, C), lambda i: (i, 0)),
        out_shape=jax.ShapeDtypeStruct((R, C), F32),
    )(x, rgsh)


def _edge_stage1(gsrc, gdst, ev, w1p, gatw, pk):
    E0, nin = gsrc.shape
    ein = ev.shape[1]
    BE = _pick(E0, [6400, 4000, 1600, 800, 400, 80, 8])
    grid = E0 // BE

    def body(gs_ref, gd_ref, e_ref, w1_ref, gw_ref, p_ref, h1_ref, sc_ref, st_ref):
        @pl.when(pl.program_id(0) == 0)
        def _():
            st_ref[...] = jnp.zeros_like(st_ref)

        xs = gs_ref[...]
        xd = gd_ref[...]
        ee = e_ref[...]
        w1 = w1_ref[...]
        z = jnp.dot(xs * p_ref[0:1, :nin] + p_ref[1:2, :nin],
                    w1[0:nin], preferred_element_type=F32)
        z += jnp.dot(xd * p_ref[0:1, nin:2 * nin] + p_ref[1:2, nin:2 * nin],
                     w1[nin:2 * nin], preferred_element_type=F32)
        z += jnp.dot(ee * p_ref[0:1, 2 * nin:2 * nin + ein]
                     + p_ref[1:2, 2 * nin:2 * nin + ein],
                     w1[2 * nin:2 * nin + ein], preferred_element_type=F32)
        z += p_ref[2:3, :32]
        h1 = _leaky(z, LEAK)
        h1_ref[...] = h1
        st_ref[...] += _stat_rows([jnp.sum(h1, axis=0),
                                   jnp.sum(h1 * h1, axis=0)], 32)

        hs = jnp.dot(xs, gw_ref[...], preferred_element_type=F32)
        s = jnp.sum(hs * p_ref[3:4, :32], axis=1, keepdims=True)
        d = jnp.sum(xd * p_ref[4:5, :nin], axis=1, keepdims=True)
        ex = jnp.exp(_leaky(s + d, 0.2))
        sc_ref[...] = jnp.concatenate(
            [hs * ex, ex, jnp.zeros((BE, 7), F32)], axis=1)

    return pl.pallas_call(
        body,
        grid=(grid,),
        in_specs=[
            pl.BlockSpec((BE, nin), lambda i: (i, 0)),
            pl.BlockSpec((BE, nin), lambda i: (i, 0)),
            pl.BlockSpec((BE, ein), lambda i: (i, 0)),
            pl.BlockSpec(w1p.shape, lambda i: (0, 0)),
            pl.BlockSpec((nin, 32), lambda i: (0, 0)),
            pl.BlockSpec((8, 128), lambda i: (0, 0)),
        ],
        out_specs=[
            pl.BlockSpec((BE, 32), lambda i: (i, 0)),
            pl.BlockSpec((BE, 40), lambda i: (i, 0)),
            pl.BlockSpec((8, 32), lambda i: (0, 0)),
        ],
        out_shape=[
            jax.ShapeDtypeStruct((E0, 32), F32),
            jax.ShapeDtypeStruct((E0, 40), F32),
            jax.ShapeDtypeStruct((8, 32), F32),
        ],
    )(gsrc, gdst, ev, w1p, gatw, pk)


def _edge_dense(h, w, pk, slope, with_stats):
    E0, cin = h.shape
    cout = w.shape[1]
    BE = _pick(E0, [6400, 4000, 1600, 800, 400, 80, 8])
    grid = E0 // BE

    def body(h_ref, w_ref, p_ref, o_ref, st_ref=None):
        if st_ref is not None:
            @pl.when(pl.program_id(0) == 0)
            def _():
                st_ref[...] = jnp.zeros_like(st_ref)

        hb = h_ref[...] * p_ref[0:1, :cin] + p_ref[1:2, :cin]
        z = jnp.dot(hb, w_ref[...], preferred_element_type=F32) + p_ref[2:3, :cout]
        if slope is not None:
            z = _leaky(z, slope)
        o_ref[...] = z
        if st_ref is not None:
            st_ref[...] += _stat_rows([jnp.sum(z, axis=0),
                                       jnp.sum(z * z, axis=0)], cout)

    out_specs = [pl.BlockSpec((BE, cout), lambda i: (i, 0))]
    out_shape = [jax.ShapeDtypeStruct((E0, cout), F32)]
    if with_stats:
        out_specs.append(pl.BlockSpec((8, cout), lambda i: (0, 0)))
        out_shape.append(jax.ShapeDtypeStruct((8, cout), F32))
    res = pl.pallas_call(
        body,
        grid=(grid,),
        in_specs=[
            pl.BlockSpec((BE, cin), lambda i: (i, 0)),
            pl.BlockSpec(w.shape, lambda i: (0, 0)),
            pl.BlockSpec((8, 128), lambda i: (0, 0)),
        ],
        out_specs=out_specs,
        out_shape=out_shape,
    )(h, w, pk)
    return res if with_stats else (res[0], None)


def _finalize(acc, xbn, gatw, pk, nodewp):
    HN0, nin = xbn.shape
    BR = _pick(HN0, [5000, 4000, 2000, 1000, 200, 8])
    grid = HN0 // BR

    def body(a_ref, x_ref, gw_ref, p_ref, *rest):
        if nodewp is not None:
            nw_ref, xo_ref, po_ref = rest
        else:
            (xo_ref,) = rest
        xb = x_ref[...]
        h = jnp.dot(xb, gw_ref[...], preferred_element_type=F32)
        s = jnp.sum(h * p_ref[0:1, :32], axis=1, keepdims=True)
        d = jnp.sum(h * p_ref[1:2, :32], axis=1, keepdims=True)
        ex = jnp.exp(_leaky(s + d, 0.2))
        num = a_ref[:, :32] + ex * h
        den = a_ref[:, 32:33] + ex
        xg = num / (den + 1e-16) + p_ref[2:3, :32]
        xn = _leaky(xg, LEAK)
        xo_ref[...] = xn
        if nodewp is not None:
            po_ref[...] = (jnp.dot(xn, nw_ref[...], preferred_element_type=F32)
                           + p_ref[3:4, :8])

    in_specs = [
        pl.BlockSpec((BR, 40), lambda i: (i, 0)),
        pl.BlockSpec((BR, nin), lambda i: (i, 0)),
        pl.BlockSpec((nin, 32), lambda i: (0, 0)),
        pl.BlockSpec((8, 128), lambda i: (0, 0)),
    ]
    args = [acc, xbn, gatw, pk]
    out_specs = [pl.BlockSpec((BR, 32), lambda i: (i, 0))]
    out_shape = [jax.ShapeDtypeStruct((HN0, 32), F32)]
    if nodewp is not None:
        in_specs.append(pl.BlockSpec((32, 8), lambda i: (0, 0)))
        args.append(nodewp)
        out_specs.append(pl.BlockSpec((BR, 8), lambda i: (i, 0)))
        out_shape.append(jax.ShapeDtypeStruct((HN0, 8), F32))
    res = pl.pallas_call(
        body,
        grid=(grid,),
        in_specs=in_specs,
        out_specs=out_specs,
        out_shape=out_shape,
    )(*args)
    return res if nodewp is not None else (res[0], None)


def _remap(dst2, hq, trash):
    R, C = dst2.shape
    BR = _pick(R, [8, 4, 2, 1])
    grid = R // BR

    def body(d_ref, *outs):
        d = d_ref[...]
        for q in range(4):
            lo, hi = q * hq, (q + 1) * hq
            outs[q][...] = jnp.where((d >= lo) & (d < hi), d - lo, trash)

    spec = pl.BlockSpec((BR, C), lambda i: (i, 0))
    sh = jax.ShapeDtypeStruct((R, C), I32)
    return pl.pallas_call(
        body,
        grid=(grid,),
        in_specs=[spec],
        out_specs=[spec] * 4,
        out_shape=[sh] * 4,
    )(dst2)


@functools.lru_cache(maxsize=None)
def _sc_gather_kernel(E0, C):
    ew = E0 // SC_NW
    gb = _pick(ew, [400, 200, 8])
    iters = ew // gb
    mesh = plsc.VectorSubcoreMesh(core_axis_name="c", subcore_axis_name="s")

    @functools.partial(
        pl.kernel,
        mesh=mesh,
        compiler_params=pltpu.CompilerParams(use_tc_tiling_on_sc=False),
        out_type=[
            jax.ShapeDtypeStruct((E0, C), F32),
            jax.ShapeDtypeStruct((E0, C), F32),
        ],
        scratch_types=[
            pltpu.VMEM((gb,), I32),
            pltpu.VMEM((gb, C), F32),
            pltpu.SemaphoreType.DMA,
        ],
    )
    def k(x_hbm, s_hbm, d_hbm, gs_hbm, gd_hbm, idx_v, rows_v, sem):
        wid = lax.axis_index("s") * SC_NC + lax.axis_index("c")
        base = wid * ew

        @pl.loop(0, iters)
        def _(i):
            off = base + i * gb
            pltpu.sync_copy(s_hbm.at[pl.ds(off, gb)], idx_v)
            pltpu.async_copy(x_hbm.at[idx_v], rows_v, sem).wait()
            pltpu.sync_copy(rows_v, gs_hbm.at[pl.ds(off, gb)])
            pltpu.sync_copy(d_hbm.at[pl.ds(off, gb)], idx_v)
            pltpu.async_copy(x_hbm.at[idx_v], rows_v, sem).wait()
            pltpu.sync_copy(rows_v, gd_hbm.at[pl.ds(off, gb)])

    return k


def _sc_gather(xbn, src, dst):
    return _sc_gather_kernel(src.shape[0], xbn.shape[1])(xbn, src, dst)


@functools.lru_cache(maxsize=None)
def _sc_scatter_kernel(E0, W, hp):
    es = E0 // SC_NS
    sb = _pick(es, [200, 8])
    iters = es // sb
    assert iters % 4 == 0
    rps = hp // SC_NS
    mesh = plsc.VectorSubcoreMesh(core_axis_name="c", subcore_axis_name="s")

    @functools.partial(
        pl.kernel,
        mesh=mesh,
        compiler_params=pltpu.CompilerParams(use_tc_tiling_on_sc=False),
        out_type=[
            jax.ShapeDtypeStruct((hp, W), F32),
            jax.ShapeDtypeStruct((hp, W), F32),
        ],
        scratch_types=[
            pltpu.VMEM((4, sb), I32),
            pltpu.VMEM((4, sb, W), F32),
            pltpu.VMEM_SHARED((hp, W), F32),
            pltpu.SemaphoreType.DMA((4,)),
            pltpu.SemaphoreType.DMA((4,)),
            pltpu.SemaphoreType.DMA((4,)),
        ],
    )
    def k(r_hbm, i0_hbm, i1_hbm, z_hbm, a0_hbm, a1_hbm, idx_v, rows_v, shared,
          lis, lrs, sss):
        cid = lax.axis_index("c")
        sid = lax.axis_index("s")

        @pl.when(sid == 0)
        def _():
            pltpu.sync_copy(z_hbm, shared)

        plsc.subcore_barrier()

        def run(ix_hbm):
            base = sid * es

            def load(j, b):
                off = base + j * sb
                pltpu.async_copy(ix_hbm.at[pl.ds(off, sb)], idx_v.at[b],
                                 lis.at[b])
                pltpu.async_copy(r_hbm.at[pl.ds(off, sb)], rows_v.at[b],
                                 lrs.at[b])

            def wait_load(b):
                pltpu.make_async_copy(ix_hbm.at[pl.ds(0, sb)], idx_v.at[b],
                                      lis.at[b]).wait()
                pltpu.make_async_copy(r_hbm.at[pl.ds(0, sb)], rows_v.at[b],
                                      lrs.at[b]).wait()

            def wait_scat(b):
                pltpu.make_async_copy(rows_v.at[b], shared.at[idx_v.at[b]],
                                      sss.at[b]).wait()

            load(0, 0)
            load(1, 1)

            @pl.loop(0, iters, step=4)
            def _(i):
                for b4 in range(4):
                    j = i + b4
                    b = b4 % 2
                    wait_load(b)
                    pltpu.async_copy(rows_v.at[b], shared.at[idx_v.at[b]],
                                     sss.at[b], add=True)
                    wait_scat(b)

                    @pl.when(j + 2 < iters)
                    def _():
                        load(j + 2, b)

        @pl.when(cid == 0)
        def _():
            run(i0_hbm)

        @pl.when(cid == 1)
        def _():
            run(i1_hbm)

        plsc.subcore_barrier()

        @pl.when(cid == 0)
        def _():
            pltpu.sync_copy(shared.at[pl.ds(sid * rps, rps)],
                            a0_hbm.at[pl.ds(sid * rps, rps)])

        @pl.when(cid == 1)
        def _():
            pltpu.sync_copy(shared.at[pl.ds(sid * rps, rps)],
                            a1_hbm.at[pl.ds(sid * rps, rps)])

    return k


def _sc_scatter(rows, idx0, idx1, zrows, hp):
    return _sc_scatter_kernel(rows.shape[0], rows.shape[1], hp)(
        rows, idx0, idx1, zrows)


def _moments(s1, s2, n):
    mu = s1 / n
    var = s2 / n - mu * mu
    return mu, var


def _affine(mu, var, g, b):
    rg = lax.rsqrt(var + BN_EPS) * g
    return rg, b - mu * rg


def kernel(node_features, edge_indices, edge_features, xbatch, params):
    del xbatch
    n = node_features.shape[0]
    e_cnt = edge_indices.shape[1]
    hq = n // 4
    rps = -(-(hq + 8) // SC_NS)
    rps += -rps % 8
    hp = rps * SC_NS
    src = edge_indices[0]
    dst = edge_indices[1]

    cols = _pick(e_cnt, [8000, 4000, 2000, 1000, 8])
    dq = [a.reshape(-1) for a in _remap(dst.reshape(-1, cols), hq, hq)]

    def scatter4(rows, idx4, z):
        a0, a1 = _sc_scatter(rows, idx4[0], idx4[1], z, hp)
        a2, a3 = _sc_scatter(rows, idx4[2], idx4[3], z + a0[0:1, 0:1] * 0.0, hp)
        return a0, a1, a2, a3

    z40 = jnp.zeros((hp, 40), F32)
    est = _colstats(edge_features)

    x = node_features
    ev = edge_features
    x_pred = None
    e_pred = None
    for r in range(NUM_MP):
        p = params['mp%d' % r]
        nin = x.shape[1]
        ein = ev.shape[1]

        nst = _colstats(x)
        mu, var = _moments(nst[0, :nin], nst[1, :nin], n)
        rg, sh = _affine(mu, var, p['bn_node_g'], p['bn_node_b'])
        xbn = _xbn(x, _pack_rows([rg, sh]))

        gsrc, gdst = _sc_gather(xbn, src, dst)

        sst = _colstats(gsrc)
        dstt = _colstats(gdst)
        mu_cat, var_cat = _moments(
            jnp.concatenate([sst[0, :nin], dstt[0, :nin], est[0, :ein]]),
            jnp.concatenate([sst[1, :nin], dstt[1, :nin], est[1, :ein]]),
            float(e_cnt))
        rg1, sh1 = _affine(mu_cat, var_cat, p['bn1_g'], p['bn1_b'])

        v_d = p['gat_W'] @ p['gat_adst']
        pk1 = _pack_rows([rg1, sh1, p['b1'], p['gat_asrc'], v_d])
        h1, scat, st1 = _edge_stage1(gsrc, gdst, ev, _pad_rows(p['w1']),
                                     p['gat_W'], pk1)

        mu2, var2 = _moments(st1[0, :32], st1[1, :32], float(e_cnt))
        rg2, sh2 = _affine(mu2, var2, p['bn2_g'], p['bn2_b'])
        h2, st2 = _edge_dense(h1, p['w2'], _pack_rows([rg2, sh2, p['b2']]),
                              LEAK, True)

        mu3, var3 = _moments(st2[0, :32], st2[1, :32], float(e_cnt))
        rg3, sh3 = _affine(mu3, var3, p['bn3_g'], p['bn3_b'])
        if r < NUM_MP - 1:
            ev, est = _edge_dense(h2, p['w3'], _pack_rows([rg3, sh3, p['b3']]),
                                  None, True)
        else:
            w3e = _pad_rows((p['w3'] @ params['edge_W']).T, 8).T
            b3e = p['b3'] @ params['edge_W'] + params['edge_b']
            ep8, _ = _edge_dense(h2, w3e, _pack_rows([rg3, sh3, b3e]),
                                 None, False)
            e_pred = ep8[:, :2]

        accs = scatter4(scat, dq, z40)

        nodewp = None
        pkf = [p['gat_asrc'], p['gat_adst'], p['gat_bias']]
        if r == NUM_MP - 1:
            nodewp = _pad_rows(params['node_W'].T, 8).T
            pkf.append(params['node_b'])
        pkf = _pack_rows(pkf)
        fins = [_finalize(accs[q][:hq], xbn[q * hq:(q + 1) * hq],
                          p['gat_W'], pkf, nodewp) for q in range(4)]
        x = jnp.concatenate([f[0] for f in fins], axis=0)
        if r == NUM_MP - 1:
            x_pred = jnp.concatenate([f[1][:, :2] for f in fins], axis=0)

    return x_pred, e_pred
